# baseline (speedup 1.0000x reference)
import functools

import jax
import jax.numpy as jnp
from jax import lax
from jax.experimental import pallas as pl
from jax.experimental.pallas import tpu as pltpu
from jax.experimental.pallas import tpu_sc as plsc

N = 10000
E = 320000
D = 128
NOISE = 64
HID = 256
NARG = 4
B = 500
NPG = 20

NC = 2
NS = 16
NW = NC * NS
K = 128
CPT = 80
BC = 40
NB = CPT // BC
NCH = NW * CPT
EP = NCH * K
NPAD = 10240
RPT = NPAD // NS
GR = NPAD // NPG

@functools.lru_cache(maxsize=1)
def _make_sc_agg():
  mesh = plsc.VectorSubcoreMesh(
      core_axis_name="c", subcore_axis_name="s", num_cores=NC, num_subcores=NS)

  @functools.partial(
      pl.kernel,
      mesh=mesh,
      out_type=jax.ShapeDtypeStruct((NC, NPAD, D), jnp.float32),
      scratch_types=[
          pltpu.VMEM((2, BC, K), jnp.int32),
          pltpu.VMEM((K, D), jnp.float32),
          pltpu.VMEM((K, D), jnp.float32),
          pltpu.VMEM_SHARED((NPAD, D), jnp.float32),
          pltpu.SemaphoreType.DMA,
          pltpu.SemaphoreType.DMA,
      ],
  )
  def _sc_agg(xsel_hbm, ei_hbm, zeros_hbm, out_hbm,
              idx_v, rows0, rows1, agg_sh, sem0, sem1):
    c = lax.axis_index("c")
    s = lax.axis_index("s")
    wid = s * NC + c

    pltpu.sync_copy(zeros_hbm, agg_sh.at[pl.ds(s * RPT, RPT)])
    plsc.subcore_barrier()

    base = wid * CPT

    def fire(j, rows, sem):
        pltpu.async_copy(xsel_hbm.at[idx_v.at[0, j]], rows, sem)

    def drain(rows, sem):
        pltpu.make_async_copy(xsel_hbm.at[pl.ds(0, K)], rows, sem).wait()

    def scat(j, rows):
        pltpu.sync_copy(rows, agg_sh.at[idx_v.at[1, j]], add=True)

    def batch(b, carry):
        pltpu.sync_copy(ei_hbm.at[:, pl.ds(base + b * BC, BC), :], idx_v)
        fire(0, rows0, sem0)

        def body(i, carry2):
            j = i * 2
            fire(j + 1, rows1, sem1)
            drain(rows0, sem0)
            scat(j, rows0)

            @pl.when(j + 2 < BC)
            def _():
                fire(j + 2, rows0, sem0)

            drain(rows1, sem1)
            scat(j + 1, rows1)
            return carry2

        lax.fori_loop(0, BC // 2, body, 0)
        return carry

    lax.fori_loop(0, NB, batch, 0)

    plsc.subcore_barrier()
    pltpu.sync_copy(agg_sh.at[pl.ds(s * RPT, RPT)],
                    out_hbm.at[c, pl.ds(s * RPT, RPT)])

  return _sc_agg


def _tc_body(xp_ref, noise_ref, aggp_ref, Ws_ref, Wn_ref, bg_ref,
             W1a_ref, W1b_ref, W1n_ref, b1_ref, W2_ref, b2_ref, out_ref):
    f32 = jnp.float32
    xb = xp_ref[...]
    agg = aggp_ref[0, :B] + aggp_ref[1, :B]
    ox = jnp.dot(xb, Ws_ref[...], preferred_element_type=f32)
    ox += jnp.dot(agg, Wn_ref[...], preferred_element_type=f32)
    ox = jnp.maximum(ox + bg_ref[...], 0.0)
    h = jnp.dot(ox, W1a_ref[0], preferred_element_type=f32)
    h += jnp.dot(xb[:, 13:19], W1b_ref[0], preferred_element_type=f32)
    h += jnp.dot(noise_ref[...], W1n_ref[0], preferred_element_type=f32)
    h = jnp.maximum(h + b1_ref[0], 0.0)
    out_ref[...] = jnp.dot(h, W2_ref[0], preferred_element_type=f32) + b2_ref[0]


_tc_mlp = pl.pallas_call(
    _tc_body,
    grid=(NPG,),
    in_specs=[
        pl.BlockSpec((B, D), lambda p: (0, p)),
        pl.BlockSpec((B, NOISE), lambda p: (0, 0)),
        pl.BlockSpec((NC, GR, D), lambda p: (0, 0, p)),
        pl.BlockSpec((D, D), lambda p: (0, 0)),
        pl.BlockSpec((D, D), lambda p: (0, 0)),
        pl.BlockSpec((1, D), lambda p: (0, 0)),
        pl.BlockSpec((1, D, HID), lambda p: (jnp.minimum(p, NARG), 0, 0)),
        pl.BlockSpec((1, 6, HID), lambda p: (jnp.minimum(p, NARG), 0, 0)),
        pl.BlockSpec((1, NOISE, HID), lambda p: (jnp.minimum(p, NARG), 0, 0)),
        pl.BlockSpec((1, 1, HID), lambda p: (jnp.minimum(p, NARG), 0, 0)),
        pl.BlockSpec((1, HID, D), lambda p: (jnp.minimum(p, NARG), 0, 0)),
        pl.BlockSpec((1, 1, D), lambda p: (jnp.minimum(p, NARG), 0, 0)),
    ],
    out_specs=pl.BlockSpec((B, D), lambda p: (0, p)),
    out_shape=jax.ShapeDtypeStruct((B, NPG * D), jnp.float32),
)


def kernel(x, edge_index, ptr, noise, Ws, Wn, bg, aW1, ab1, aW2, ab2,
           sW1, sb1, sW2, sb2):
    del ptr
    xsel = x[:, :D]

    pad_iota = jnp.arange(EP - E, dtype=jnp.int32)
    pad = jnp.stack([pad_iota % N, N + pad_iota % (NPAD - N)])
    eip = jnp.concatenate([edge_index, pad], axis=1).reshape(2, NCH, K)
    zeros = jnp.zeros((RPT, D), jnp.float32)

    aggp = _make_sc_agg()(xsel, eip, zeros)
    aggp = aggp.reshape(NC, GR, NPG * D)

    NU = NARG + 1
    W1 = jnp.concatenate([aW1, sW1[None]])
    W1a = W1[:, :D, :]
    W1b = W1[:, D:D + 6, :]
    W1n = W1[:, D + 6:, :]
    b1 = jnp.concatenate([ab1, sb1[None]])
    W2 = jnp.concatenate([aW2, sW2[None]])
    b2 = jnp.concatenate([ab2, sb2[None]])

    outp = _tc_mlp(xsel.reshape(B, NPG * D), noise, aggp, Ws, Wn,
                   bg.reshape(1, D), W1a, W1b, W1n, b1.reshape(NU, 1, HID),
                   W2, b2.reshape(NU, 1, D))
    return outp.reshape(N, D)


# speedup vs baseline: 1.0624x; 1.0624x over previous
import functools

import jax
import jax.numpy as jnp
from jax import lax
from jax.experimental import pallas as pl
from jax.experimental.pallas import tpu as pltpu
from jax.experimental.pallas import tpu_sc as plsc

N = 10000
E = 320000
D = 128
NOISE = 64
HID = 256
NARG = 4
B = 500
NPG = 20

NC = 2
NS = 16
NW = NC * NS
K = 128
CPT = 80
BC = 40
NB = CPT // BC
NCH = NW * CPT
EP = NCH * K
B2 = 512
NPAD = NPG * B2
RPT = NPAD // NS
MLP_IN = D + NOISE + 6

@functools.lru_cache(maxsize=1)
def _make_sc_agg():
  mesh = plsc.VectorSubcoreMesh(
      core_axis_name="c", subcore_axis_name="s", num_cores=NC, num_subcores=NS)

  @functools.partial(
      pl.kernel,
      mesh=mesh,
      out_type=jax.ShapeDtypeStruct((NC, NPAD, D), jnp.float32),
      scratch_types=[
          pltpu.VMEM((2, BC, K), jnp.int32),
          pltpu.VMEM((K,), jnp.int32),
          pltpu.VMEM((K, D), jnp.float32),
          pltpu.VMEM((K, D), jnp.float32),
          pltpu.VMEM_SHARED((NPAD, D), jnp.float32),
          pltpu.SemaphoreType.DMA,
          pltpu.SemaphoreType.DMA,
      ],
  )
  def _sc_agg(xsel_hbm, ei_hbm, zeros_hbm, out_hbm,
              idx_v, dstr_v, rows0, rows1, agg_sh, sem0, sem1):
    c = lax.axis_index("c")
    s = lax.axis_index("s")
    wid = s * NC + c

    pltpu.sync_copy(zeros_hbm, agg_sh.at[pl.ds(s * RPT, RPT)])
    plsc.subcore_barrier()

    base = wid * CPT

    def fire(j, rows, sem):
        pltpu.async_copy(xsel_hbm.at[idx_v.at[0, j]], rows, sem)

    def drain(rows, sem):
        pltpu.make_async_copy(xsel_hbm.at[pl.ds(0, K)], rows, sem).wait()

    def scat(j, rows):
        for l in range(K // 16):
            d = idx_v[1, j, pl.ds(l * 16, 16)]
            q = lax.shift_right_logical(d * 3277, 16)
            r = d - q * NPG
            dstr_v[pl.ds(l * 16, 16)] = lax.shift_left(r, 9) + q
        pltpu.sync_copy(rows, agg_sh.at[dstr_v], add=True)

    def batch(b, carry):
        pltpu.sync_copy(ei_hbm.at[:, pl.ds(base + b * BC, BC), :], idx_v)
        fire(0, rows0, sem0)

        def body(i, carry2):
            j = i * 2
            fire(j + 1, rows1, sem1)
            drain(rows0, sem0)
            scat(j, rows0)

            @pl.when(j + 2 < BC)
            def _():
                fire(j + 2, rows0, sem0)

            drain(rows1, sem1)
            scat(j + 1, rows1)
            return carry2

        lax.fori_loop(0, BC // 2, body, 0)
        return carry

    lax.fori_loop(0, NB, batch, 0)

    plsc.subcore_barrier()
    pltpu.sync_copy(agg_sh.at[pl.ds(s * RPT, RPT)],
                    out_hbm.at[c, pl.ds(s * RPT, RPT)])

  return _sc_agg


def _tc_body(xp_ref, noise_ref, aggp_ref, Wsn_ref, bg_ref,
             W1_ref, b1_ref, W2_ref, b2_ref, out_ref):
    f32 = jnp.float32
    xb = xp_ref[0]
    agg = aggp_ref[0, 0, :B] + aggp_ref[1, 0, :B]
    xa = jnp.concatenate([xb, agg], axis=1)
    ox = jnp.dot(xa, Wsn_ref[...], preferred_element_type=f32)
    ox = jnp.maximum(ox + bg_ref[...], 0.0)
    hin = jnp.concatenate([ox, noise_ref[...], xb[:, 13:19]], axis=1)
    h = jnp.dot(hin, W1_ref[0], preferred_element_type=f32)
    h = jnp.maximum(h + b1_ref[0], 0.0)
    out_ref[...] = jnp.dot(h, W2_ref[0], preferred_element_type=f32) + b2_ref[0]


_tc_mlp = pl.pallas_call(
    _tc_body,
    grid=(NPG,),
    in_specs=[
        pl.BlockSpec((1, B, D), lambda p: (p, 0, 0)),
        pl.BlockSpec((B, NOISE), lambda p: (0, 0)),
        pl.BlockSpec((NC, 1, B2, D), lambda p: (0, p, 0, 0)),
        pl.BlockSpec((2 * D, D), lambda p: (0, 0)),
        pl.BlockSpec((1, D), lambda p: (0, 0)),
        pl.BlockSpec((1, MLP_IN, HID), lambda p: (jnp.minimum(p, NARG), 0, 0)),
        pl.BlockSpec((1, 1, HID), lambda p: (jnp.minimum(p, NARG), 0, 0)),
        pl.BlockSpec((1, HID, D), lambda p: (jnp.minimum(p, NARG), 0, 0)),
        pl.BlockSpec((1, 1, D), lambda p: (jnp.minimum(p, NARG), 0, 0)),
    ],
    out_specs=pl.BlockSpec((B, D), lambda p: (0, p)),
    out_shape=jax.ShapeDtypeStruct((B, NPG * D), jnp.float32),
)


def kernel(x, edge_index, ptr, noise, Ws, Wn, bg, aW1, ab1, aW2, ab2,
           sW1, sb1, sW2, sb2):
    del ptr
    xsel = x[:, :D]

    pad_iota = jnp.arange(EP - E, dtype=jnp.int32)
    pad = jnp.stack([pad_iota % N, N + pad_iota % (NPAD - N)])
    eip = jnp.concatenate([edge_index, pad], axis=1).reshape(2, NCH, K)
    zeros = jnp.zeros((RPT, D), jnp.float32)

    aggp = _make_sc_agg()(xsel, eip, zeros)
    aggp = aggp.reshape(NC, NPG, B2, D)
    xp = xsel.reshape(B, NPG, D).transpose(1, 0, 2)

    NU = NARG + 1
    W1 = jnp.concatenate([aW1, sW1[None]])
    W1r = jnp.concatenate([W1[:, :D, :], W1[:, D + 6:, :], W1[:, D:D + 6, :]],
                          axis=1)
    b1 = jnp.concatenate([ab1, sb1[None]])
    W2 = jnp.concatenate([aW2, sW2[None]])
    b2 = jnp.concatenate([ab2, sb2[None]])
    Wsn = jnp.concatenate([Ws, Wn], axis=0)

    outp = _tc_mlp(xp, noise, aggp, Wsn, bg.reshape(1, D),
                   W1r, b1.reshape(NU, 1, HID), W2, b2.reshape(NU, 1, D))
    return outp.reshape(N, D)
